# baseline (speedup 1.0000x reference)
import jax
import jax.numpy as jnp
from jax.experimental import pallas as pl


def kernel(x, tables):
    cont = x[:, :13].astype(jnp.float32)
    offs = (jnp.arange(26, dtype=jnp.int32) * 100000)[None, :]
    idx = x[:, 13:] + offs
    tab = tables.reshape(26 * 100000, 16)
    emb = jnp.take(tab, idx.reshape(-1), axis=0).reshape(16384, 416)
    return jnp.concatenate([cont, emb], axis=1)


# speedup vs baseline: 5.2305x; 5.2305x over previous
import functools

import jax
import jax.numpy as jnp
from jax import lax
from jax.experimental import pallas as pl
from jax.experimental.pallas import tpu as pltpu
from jax.experimental.pallas import tpu_sc as plsc

_INPUT_DIM = 39
_N_CAT = 26
_VOCAB = 100000
_EMB = 16
_BATCH = 16384
_N_CONT = _INPUT_DIM - _N_CAT

_NC = 2
_NS = 16
_NW = _NC * _NS

_B_PER_W = _BATCH // _NW
_CHUNK = 64
_N_CHUNKS = _B_PER_W // _CHUNK
_ROWS = _CHUNK * _N_CAT
_GB = 128

_L = 16


@functools.partial(
    pl.kernel,
    mesh=plsc.VectorSubcoreMesh(core_axis_name="c", subcore_axis_name="s"),
    out_type=(
        jax.ShapeDtypeStruct((_BATCH * _N_CAT, _EMB), jnp.float32),
        jax.ShapeDtypeStruct((_BATCH, _N_CONT), jnp.float32),
    ),
    scratch_types=[
        pltpu.VMEM((_INPUT_DIM, _B_PER_W), jnp.int32),
        pltpu.VMEM((_ROWS,), jnp.int32),
        pltpu.VMEM((_ROWS, _EMB), jnp.float32),
        pltpu.VMEM((_CHUNK, _N_CONT), jnp.float32),
        pltpu.SemaphoreType.DMA,
    ],
    compiler_params=pltpu.CompilerParams(
        use_tc_tiling_on_sc=False, needs_layout_passes=False
    ),
)
def _sc_embed(xt_hbm, tab_hbm, emb_hbm, cont_hbm,
              xt_v, idx_v, rows_v, cont_v, sem):
    wid = lax.axis_index("s") * _NC + lax.axis_index("c")
    w0 = wid * _B_PER_W
    pltpu.sync_copy(xt_hbm.at[:, pl.ds(w0, _B_PER_W)], xt_v)
    iota = lax.iota(jnp.int32, _L)

    def chunk_body(c, carry):
        for j in range(_N_CAT):
            for g in range(_CHUNK // _L):
                r = xt_v[_N_CONT + j, pl.ds(c * _CHUNK + g * _L, _L)]
                pos = (g * _L + iota) * _N_CAT + j
                plsc.store_scatter(idx_v, [pos], r + j * _VOCAB)

        copies = [
            pltpu.async_copy(
                tab_hbm.at[idx_v.at[pl.ds(k * _GB, _GB)]],
                rows_v.at[pl.ds(k * _GB, _GB)],
                sem,
            )
            for k in range(_ROWS // _GB)
        ]

        for col in range(_N_CONT):
            for g in range(_CHUNK // _L):
                vals = xt_v[col, pl.ds(c * _CHUNK + g * _L, _L)]
                plsc.store_scatter(cont_v, [g * _L + iota, col + iota * 0],
                                   vals.astype(jnp.float32))

        for cp in copies:
            cp.wait()

        b0 = w0 + c * _CHUNK
        pltpu.sync_copy(rows_v, emb_hbm.at[pl.ds(b0 * _N_CAT, _ROWS)])
        pltpu.sync_copy(cont_v, cont_hbm.at[pl.ds(b0, _CHUNK)])
        return carry

    lax.fori_loop(0, _N_CHUNKS, chunk_body, 0)


def kernel(x, tables):
    tab = tables.reshape(_N_CAT * _VOCAB, _EMB)
    emb, cont = _sc_embed(x.T, tab)
    return jnp.concatenate([cont, emb.reshape(_BATCH, _N_CAT * _EMB)], axis=1)
